# Initial kernel scaffold; baseline (speedup 1.0000x reference)
import jax
import jax.numpy as jnp
from jax.experimental import pallas as pl


def kernel(lf_fea, W1, W2):
    raise NotImplementedError("write your pallas kernel here")


# speedup vs baseline: 2.4444x; 2.4444x over previous
import jax
import jax.numpy as jnp
from jax.experimental import pallas as pl

_AN2 = 25
_CH = 64
_K = 6
_PS = 4
_CAND = 9
_STRIDE = 2
_HW = 48
_PN = _HW // _PS
_D = _CH * _PS * _PS


def _patches(x, N):
    x = x.reshape(N, _AN2, _CH, _PN, _PS, _PN, _PS)
    x = x.transpose(0, 3, 5, 2, 4, 6, 1)
    return x.reshape(N, _PN, _PN, _D, _AN2)


def _match_kernel(*refs):
    cand_refs = refs[:_CAND]
    csq_refs = refs[_CAND:2 * _CAND]
    w1_ref = refs[2 * _CAND]
    out_ref = refs[2 * _CAND + 1]
    c_all = jnp.concatenate([r[0, 0, 0] for r in cand_refs], axis=1)
    csq = jnp.concatenate([r[0, 0, 0] for r in csq_refs], axis=1)
    w1t = w1_ref[...]
    ncand = _CAND * _AN2
    xq = c_all[:, :_AN2].T
    scores = -2.0 * jnp.dot(xq, c_all, preferred_element_type=jnp.float32) + csq
    iota = jax.lax.broadcasted_iota(jnp.int32, scores.shape, 1)
    masks = []
    s = scores
    for _ in range(_K):
        m = jnp.min(s, axis=1, keepdims=True)
        first = jnp.min(jnp.where(s == m, iota, ncand), axis=1, keepdims=True)
        mk = iota == first
        masks.append(mk.astype(jnp.float32))
        s = jnp.where(mk, jnp.float32(3.4e38), s)
    g = jnp.stack(masks, axis=1).reshape(_AN2 * _K, ncand)
    sel = jax.lax.dot_general(c_all, g, (((1,), (1,)), ((), ())),
                              preferred_element_type=jnp.float32)
    x = sel.reshape(_CH, _PS * _PS, _AN2, _K)
    x = x.transpose(2, 1, 3, 0).reshape(_AN2 * _PS * _PS, _K * _CH)
    y = jnp.dot(x, w1t, preferred_element_type=jnp.float32)
    out_ref[0, 0, 0] = jnp.where(y >= 0, y, 0.1 * y)


def _conv_kernel(z_ref, w2_ref, out_ref):
    z = z_ref[0]
    acc = jnp.zeros((_CH, _HW * _HW), jnp.float32)
    for dy in range(3):
        for dx in range(3):
            w = w2_ref[:, :, dy, dx]
            patch = z[:, dy:dy + _HW, dx:dx + _HW].reshape(2 * _CH, _HW * _HW)
            acc = acc + jnp.dot(w, patch, preferred_element_type=jnp.float32)
    acc = jnp.where(acc >= 0, acc, 0.1 * acc)
    out_ref[0] = acc.reshape(_CH, _HW, _HW)


def kernel(lf_fea, W1, W2):
    B_total = lf_fea.shape[0]
    N = B_total // _AN2

    bases = {}
    for sx in (0, _STRIDE):
        for sy in (0, _STRIDE):
            src = lf_fea if (sx == 0 and sy == 0) else jnp.roll(
                lf_fea, shift=(sx, sy), axis=(2, 3))
            p = bases[(sx, sy)] = _patches(src, N)
            bases[(sx, sy, "csq")] = jnp.sum(
                p.reshape(N * _PN * _PN, _D, _AN2) ** 2, axis=-2,
                keepdims=True).reshape(N, _PN, _PN, 1, _AN2)

    cand_ops, csq_ops, cand_specs, csq_specs = [], [], [], []
    for i in range(_CAND):
        xs = (i // 3) * _STRIDE
        ys = (i % 3) * _STRIDE
        base = (xs % _PS, ys % _PS)
        dh, dw = xs // _PS, ys // _PS
        cand_ops.append(bases[base])
        csq_ops.append(bases[base + ("csq",)])

        def imap(n, ph, pw, dh=dh, dw=dw):
            return (n, (ph + _PN - dh) % _PN, (pw + _PN - dw) % _PN, 0, 0)

        cand_specs.append(pl.BlockSpec((1, 1, 1, _D, _AN2), imap))
        csq_specs.append(pl.BlockSpec((1, 1, 1, 1, _AN2), imap))

    w1t = W1.reshape(_CH, _K * _CH).T

    out1 = pl.pallas_call(
        _match_kernel,
        grid=(N, _PN, _PN),
        in_specs=cand_specs + csq_specs + [
            pl.BlockSpec((_K * _CH, _CH), lambda n, ph, pw: (0, 0))],
        out_specs=pl.BlockSpec((1, 1, 1, _AN2 * _PS * _PS, _CH),
                               lambda n, ph, pw: (n, ph, pw, 0, 0)),
        out_shape=jax.ShapeDtypeStruct((N, _PN, _PN, _AN2 * _PS * _PS, _CH),
                                       jnp.float32),
    )(*cand_ops, *csq_ops, w1t)

    sp = out1.reshape(N, _PN, _PN, _AN2, _PS, _PS, _CH)
    sp = sp.transpose(0, 3, 6, 1, 4, 2, 5).reshape(N * _AN2, _CH, _HW, _HW)

    z = jnp.concatenate([lf_fea, sp], axis=1)
    zpad = jnp.pad(z, ((0, 0), (0, 0), (1, 1), (1, 1)))
    out = pl.pallas_call(
        _conv_kernel,
        grid=(B_total,),
        in_specs=[
            pl.BlockSpec((1, 2 * _CH, _HW + 2, _HW + 2), lambda b: (b, 0, 0, 0)),
            pl.BlockSpec((_CH, 2 * _CH, 3, 3), lambda b: (0, 0, 0, 0)),
        ],
        out_specs=pl.BlockSpec((1, _CH, _HW, _HW), lambda b: (b, 0, 0, 0)),
        out_shape=jax.ShapeDtypeStruct((B_total, _CH, _HW, _HW), jnp.float32),
    )(zpad, W2)
    return out
